# baseline (speedup 1.0000x reference)
import functools

import numpy as np
import jax
import jax.numpy as jnp
from jax import lax
from jax.experimental import pallas as pl
from jax.experimental.pallas import tpu as pltpu
from jax.experimental.pallas import tpu_sc as plsc

_EPS = 1e-6
_LIMIT_A = -0.1
_LIMIT_B = 1.1
_BETA = 2.0 / 3.0
_XN = (0.0 - _LIMIT_A) / (_LIMIT_B - _LIMIT_A)
_C = float(np.log(_XN / (1.0 - _XN))) * _BETA

_N = 8192
_L = 16
_NV = _N // _L


def _round_half_even(x):
    tr = x.astype(jnp.int32)
    frac = x - tr.astype(jnp.float32)
    bump = jnp.where(frac > 0.5, jnp.int32(1),
                     jnp.where(frac == 0.5, tr & 1, jnp.int32(0)))
    return tr + bump


def _partition_round(src, dst, state, bit, k):
    start, n, count_lt, prefix = state
    nv = (n + (_L - 1)) >> 4

    def body(j, carry):
        off_lo, off_hi = carry
        v = src[pl.ds(start + j * _L, _L)]
        lane = lax.iota(jnp.int32, _L)
        valid = lane < (n - j * _L)
        is0 = (v & bit) == 0
        m0 = valid & is0
        m1 = valid & jnp.logical_not(is0)
        c0 = jnp.sum(m0.astype(jnp.int32))
        c1 = jnp.sum(m1.astype(jnp.int32))
        plsc.store_compressed(dst.at[pl.ds(off_lo, _L)], v, mask=m0)
        new_hi = off_hi - c1
        plsc.store_compressed(dst.at[pl.ds(new_hi, _L)], v, mask=m1)
        return off_lo + c0, new_hi

    off_lo, off_hi = lax.fori_loop(0, nv, body, (jnp.int32(0), jnp.int32(_N)))
    cnt0 = off_lo
    take_low = (count_lt + cnt0) >= k
    start_n = jnp.where(take_low, jnp.int32(0), off_hi)
    n_n = jnp.where(take_low, cnt0, n - cnt0)
    count_lt_n = jnp.where(take_low, count_lt, count_lt + cnt0)
    prefix_n = jnp.where(take_low, prefix, prefix | bit)
    return start_n, n_n, count_lt_n, prefix_n


def _sc_body(x_hbm, o_hbm, xv, keys, ping, pong, outv):
    wid = lax.axis_index("s") * 2 + lax.axis_index("c")

    @pl.when(wid < 3)
    def _():
        pltpu.sync_copy(x_hbm.at[wid], xv)

        def pass_a(i, accs):
            new = []
            for u in range(4):
                j = i * 4 + u
                v = xv[pl.ds(j * _L, _L)]
                e = jnp.exp(v - _C)
                s = 1.0 / (1.0 + e)
                s = jnp.clip(s, _EPS, 1.0 - _EPS)
                new.append(accs[u] + (1.0 - s))
                vc = jnp.where(v == 0.0, 0.0, v)
                b = plsc.bitcast(vc, jnp.uint32)
                flip = jnp.where((b >> 31) != 0,
                                 jnp.uint32(0xFFFFFFFF), jnp.uint32(0x80000000))
                keys[pl.ds(j * _L, _L)] = b ^ flip
            return tuple(new)

        z16 = jnp.zeros((_L,), jnp.float32)
        a0, a1, a2, a3 = lax.fori_loop(0, _NV // 4, pass_a, (z16, z16, z16, z16))
        total = jnp.sum((a0 + a1) + (a2 + a3))
        k = _round_half_even(np.float32(_N) - total)

        state = (jnp.int32(0), jnp.int32(_N), jnp.int32(0), jnp.uint32(0))
        for r in range(32):
            src = keys if r == 0 else (pong if r % 2 == 0 else ping)
            dst = ping if r % 2 == 0 else pong
            state = _partition_round(src, dst, state, jnp.uint32(1 << (31 - r)), k)
        _, _, count_lt, t_key = state
        tie_budget = k - count_lt

        def eq_body(j, off):
            v = keys[pl.ds(j * _L, _L)]
            m = v == t_key
            idxv = (lax.iota(jnp.int32, _L) + j * _L).astype(jnp.uint32)
            plsc.store_compressed(ping.at[pl.ds(off, _L)], idxv, mask=m)
            return off + jnp.sum(m.astype(jnp.int32))

        n_eq = lax.fori_loop(0, _NV, eq_body, jnp.int32(0))

        state2 = (jnp.int32(0), n_eq, jnp.int32(0), jnp.uint32(0))
        for r in range(13):
            src = ping if r % 2 == 0 else pong
            dst = pong if r % 2 == 0 else ping
            state2 = _partition_round(src, dst, state2,
                                      jnp.uint32(1 << (12 - r)), tie_budget)
        i_cut = state2[3].astype(jnp.int32)

        it_key = plsc.bitcast(
            jnp.full((_L,), t_key ^ jnp.uint32(0x80000000)), jnp.int32)

        def mask_body(j, _):
            v = keys[pl.ds(j * _L, _L)]
            ik = plsc.bitcast(v ^ jnp.uint32(0x80000000), jnp.int32)
            eq = v == t_key
            idxv = lax.iota(jnp.int32, _L) + j * _L
            zero = (ik < it_key) | (eq & (idxv <= i_cut))
            outv[pl.ds(j * _L, _L)] = jnp.where(zero, 0.0, 1.0)
            return 0

        lax.fori_loop(0, _NV, mask_body, 0)
        pltpu.sync_copy(outv, o_hbm.at[wid])


@functools.partial(jax.jit, static_argnames=())
def _run_sc(x):
    f = pl.kernel(
        _sc_body,
        out_type=jax.ShapeDtypeStruct((3, _N), jnp.float32),
        mesh=plsc.VectorSubcoreMesh(core_axis_name="c", subcore_axis_name="s"),
        scratch_types=[
            pltpu.VMEM((_N,), jnp.float32),
            pltpu.VMEM((_N,), jnp.uint32),
            pltpu.VMEM((_N + _L,), jnp.uint32),
            pltpu.VMEM((_N + _L,), jnp.uint32),
            pltpu.VMEM((_N,), jnp.float32),
        ],
        compiler_params=pltpu.CompilerParams(needs_layout_passes=False),
    )
    return f(x)


def kernel(self_att_layer_loga, cross_att_layer_loga, ffn_layer_loga):
    x = jnp.stack([self_att_layer_loga, cross_att_layer_loga, ffn_layer_loga])
    out = _run_sc(x)
    return (out[0], out[1], out[2])


# speedup vs baseline: 1.6446x; 1.6446x over previous
import functools

import numpy as np
import jax
import jax.numpy as jnp
from jax import lax
from jax.experimental import pallas as pl
from jax.experimental.pallas import tpu as pltpu
from jax.experimental.pallas import tpu_sc as plsc

_EPS = 1e-6
_LIMIT_A = -0.1
_LIMIT_B = 1.1
_BETA = 2.0 / 3.0
_XN = (0.0 - _LIMIT_A) / (_LIMIT_B - _LIMIT_A)
_C = float(np.log(_XN / (1.0 - _XN))) * _BETA

_N = 8192
_L = 16
_NV = _N // _L
_NB = 256
_BIG = np.int32(0x7FFFFFFF)


def _round_half_even(x):
    tr = x.astype(jnp.int32)
    frac = x - tr.astype(jnp.float32)
    bump = jnp.where(frac > 0.5, jnp.int32(1),
                     jnp.where(frac == 0.5, tr & 1, jnp.int32(0)))
    return tr + bump


def _pick_bucket(hist, kk):
    lane = lax.iota(jnp.int32, _L)
    z16 = jnp.zeros((_L,), jnp.int32)

    def body(g, carry):
        run_min, cnt_carry = carry
        acc = z16
        for l in range(_L):
            sl = pl.ds(l * _NB + g * _L, _L)
            acc = acc + hist[sl]
            hist[sl] = z16
        cum = plsc.cumsum(acc) + cnt_carry
        found = cum >= kk
        cand = jnp.where(found, ((g * _L + lane) << 14) | (cum - acc), _BIG)
        return jnp.minimum(run_min, cand), cnt_carry + jnp.sum(acc)

    run_min, _ = lax.fori_loop(0, _NB // _L, body,
                               (jnp.full((_L,), _BIG), jnp.int32(0)))
    cm = jnp.min(run_min)
    return cm >> 14, cm & jnp.int32(0x3FFF)


def _sc_body(x_hbm, o_hbm, xv, keys, hist, outv):
    wid = lax.axis_index("s") * 2 + lax.axis_index("c")
    lane = lax.iota(jnp.int32, _L)
    lane_base = lane * _NB
    ones_i32 = jnp.ones((_L,), jnp.int32)
    z16_i32 = jnp.zeros((_L,), jnp.int32)

    @pl.when(wid < 3)
    def _():
        pltpu.sync_copy(x_hbm.at[wid], xv)

        def zero_body(j, _):
            hist[pl.ds(j * _L, _L)] = z16_i32
            return 0

        lax.fori_loop(0, (_NB * _L) // _L, zero_body, 0)

        def pass_a(i, accs):
            new = []
            for u in range(4):
                j = i * 4 + u
                v = xv[pl.ds(j * _L, _L)]
                e = jnp.exp(v - _C)
                s = 1.0 / (1.0 + e)
                s = jnp.clip(s, _EPS, 1.0 - _EPS)
                new.append(accs[u] + (1.0 - s))
                vc = jnp.where(v == 0.0, 0.0, v)
                b = plsc.bitcast(vc, jnp.uint32)
                flip = jnp.where((b >> 31) != 0,
                                 jnp.uint32(0xFFFFFFFF), jnp.uint32(0x80000000))
                keys[pl.ds(j * _L, _L)] = b ^ flip
            return tuple(new)

        z16f = jnp.zeros((_L,), jnp.float32)
        a0, a1, a2, a3 = lax.fori_loop(0, _NV // 4, pass_a,
                                       (z16f, z16f, z16f, z16f))
        total = jnp.sum((a0 + a1) + (a2 + a3))
        k = _round_half_even(np.float32(_N) - total)

        count_lt = jnp.int32(0)
        prefix = jnp.uint32(0)
        for lvl in range(4):
            shift = jnp.uint32(24 - 8 * lvl)

            def scan_body(j, _, shift=shift, lvl=lvl, prefix=prefix):
                v = keys[pl.ds(j * _L, _L)]
                b = ((v >> shift) & jnp.uint32(0xFF)).astype(jnp.int32)
                pos = lane_base + b
                if lvl == 0:
                    plsc.addupdate_scatter(hist, [pos], ones_i32)
                else:
                    m = (v >> (shift + jnp.uint32(8))) == prefix
                    plsc.addupdate_scatter(hist, [pos], ones_i32, mask=m)
                return 0

            lax.fori_loop(0, _NV, scan_body, 0)
            bucket, below = _pick_bucket(hist, k - count_lt)
            count_lt = count_lt + below
            prefix = (prefix << jnp.uint32(8)) | bucket.astype(jnp.uint32)
        t_key = prefix
        tie_budget = k - count_lt

        def tie_a(j, _):
            v = keys[pl.ds(j * _L, _L)]
            m = v == t_key
            idxv = lane + j * _L
            plsc.addupdate_scatter(hist, [lane_base + (idxv >> 5)], ones_i32,
                                   mask=m)
            return 0

        lax.fori_loop(0, _NV, tie_a, 0)
        buck_a, below_a = _pick_bucket(hist, tie_budget)

        def tie_b(j, _):
            v = keys[pl.ds(j * _L, _L)]
            idxv = lane + j * _L
            m = (v == t_key) & ((idxv >> 5) == buck_a)
            plsc.addupdate_scatter(hist, [lane_base + (idxv & 31)], ones_i32,
                                   mask=m)
            return 0

        lax.fori_loop(0, _NV, tie_b, 0)
        buck_b, _unused = _pick_bucket(hist, tie_budget - below_a)
        i_cut = buck_a * 32 + buck_b

        it_key = plsc.bitcast(
            jnp.full((_L,), t_key ^ jnp.uint32(0x80000000)), jnp.int32)

        def mask_body(j, _):
            v = keys[pl.ds(j * _L, _L)]
            ik = plsc.bitcast(v ^ jnp.uint32(0x80000000), jnp.int32)
            eq = v == t_key
            idxv = lane + j * _L
            zero = (ik < it_key) | (eq & (idxv <= i_cut))
            outv[pl.ds(j * _L, _L)] = jnp.where(zero, 0.0, 1.0)
            return 0

        lax.fori_loop(0, _NV, mask_body, 0)
        pltpu.sync_copy(outv, o_hbm.at[wid])


@jax.jit
def _run_sc(x):
    f = pl.kernel(
        _sc_body,
        out_type=jax.ShapeDtypeStruct((3, _N), jnp.float32),
        mesh=plsc.VectorSubcoreMesh(core_axis_name="c", subcore_axis_name="s"),
        scratch_types=[
            pltpu.VMEM((_N,), jnp.float32),
            pltpu.VMEM((_N,), jnp.uint32),
            pltpu.VMEM((_NB * _L,), jnp.int32),
            pltpu.VMEM((_N,), jnp.float32),
        ],
        compiler_params=pltpu.CompilerParams(needs_layout_passes=False),
    )
    return f(x)


def kernel(self_att_layer_loga, cross_att_layer_loga, ffn_layer_loga):
    x = jnp.stack([self_att_layer_loga, cross_att_layer_loga, ffn_layer_loga])
    out = _run_sc(x)
    return (out[0], out[1], out[2])


# speedup vs baseline: 1.6966x; 1.0316x over previous
import functools

import numpy as np
import jax
import jax.numpy as jnp
from jax import lax
from jax.experimental import pallas as pl
from jax.experimental.pallas import tpu as pltpu
from jax.experimental.pallas import tpu_sc as plsc

_EPS = 1e-6
_LIMIT_A = -0.1
_LIMIT_B = 1.1
_BETA = 2.0 / 3.0
_XN = (0.0 - _LIMIT_A) / (_LIMIT_B - _LIMIT_A)
_C = float(np.log(_XN / (1.0 - _XN))) * _BETA

_N = 8192
_L = 16
_NV = _N // _L
_NB = 256
_BIG = np.int32(0x7FFFFFFF)


def _round_half_even(x):
    tr = x.astype(jnp.int32)
    frac = x - tr.astype(jnp.float32)
    bump = jnp.where(frac > 0.5, jnp.int32(1),
                     jnp.where(frac == 0.5, tr & 1, jnp.int32(0)))
    return tr + bump


def _pick_bucket(hist, kk):
    lane = lax.iota(jnp.int32, _L)
    z16 = jnp.zeros((_L,), jnp.int32)

    def body(g, carry):
        run_min, cnt_carry = carry
        acc = z16
        for l in range(_L):
            sl = pl.ds(l * _NB + g * _L, _L)
            acc = acc + hist[sl]
            hist[sl] = z16
        cum = plsc.cumsum(acc) + cnt_carry
        found = cum >= kk
        cand = jnp.where(found, ((g * _L + lane) << 14) | (cum - acc), _BIG)
        return jnp.minimum(run_min, cand), cnt_carry + jnp.sum(acc)

    run_min, _ = lax.fori_loop(0, _NB // _L, body,
                               (jnp.full((_L,), _BIG), jnp.int32(0)))
    cm = jnp.min(run_min)
    return cm >> 14, cm & jnp.int32(0x3FFF)


def _sc_body(x_hbm, o_hbm, xv, keys, hist, outv):
    wid = lax.axis_index("s") * 2 + lax.axis_index("c")
    lane = lax.iota(jnp.int32, _L)
    lane_base = lane * _NB
    ones_i32 = jnp.ones((_L,), jnp.int32)
    z16_i32 = jnp.zeros((_L,), jnp.int32)

    @pl.when(wid < 3)
    def _():
        pltpu.sync_copy(x_hbm.at[wid], xv)

        def zero_body(i, _):
            for u in range(8):
                hist[pl.ds((i * 8 + u) * _L, _L)] = z16_i32
            return 0

        lax.fori_loop(0, _NB // 8, zero_body, 0)

        def pass_a(i, accs):
            new = []
            for u in range(4):
                j = i * 4 + u
                v = xv[pl.ds(j * _L, _L)]
                e = jnp.exp(v - _C)
                s = 1.0 / (1.0 + e)
                s = jnp.clip(s, _EPS, 1.0 - _EPS)
                new.append(accs[u] + (1.0 - s))
                vc = jnp.where(v == 0.0, 0.0, v)
                b = plsc.bitcast(vc, jnp.uint32)
                flip = jnp.where((b >> 31) != 0,
                                 jnp.uint32(0xFFFFFFFF), jnp.uint32(0x80000000))
                uk = b ^ flip
                keys[pl.ds(j * _L, _L)] = uk
                pos = lane_base + (uk >> 24).astype(jnp.int32)
                plsc.addupdate_scatter(hist, [pos], ones_i32)
            return tuple(new)

        z16f = jnp.zeros((_L,), jnp.float32)
        a0, a1, a2, a3 = lax.fori_loop(0, _NV // 4, pass_a,
                                       (z16f, z16f, z16f, z16f))
        total = jnp.sum((a0 + a1) + (a2 + a3))
        k = _round_half_even(np.float32(_N) - total)

        bucket, below = _pick_bucket(hist, k)
        count_lt = below
        prefix = bucket.astype(jnp.uint32)
        for lvl in range(1, 4):
            shift = jnp.uint32(24 - 8 * lvl)

            def scan_body(i, _, shift=shift, prefix=prefix):
                for u in range(8):
                    j = i * 8 + u
                    v = keys[pl.ds(j * _L, _L)]
                    b = ((v >> shift) & jnp.uint32(0xFF)).astype(jnp.int32)
                    m = (v >> (shift + jnp.uint32(8))) == prefix
                    plsc.addupdate_scatter(hist, [lane_base + b], ones_i32,
                                           mask=m)
                return 0

            lax.fori_loop(0, _NV // 8, scan_body, 0)
            bucket, below = _pick_bucket(hist, k - count_lt)
            count_lt = count_lt + below
            prefix = (prefix << jnp.uint32(8)) | bucket.astype(jnp.uint32)
        t_key = prefix
        tie_budget = k - count_lt

        def tie_a(i, _):
            for u in range(8):
                j = i * 8 + u
                v = keys[pl.ds(j * _L, _L)]
                m = v == t_key
                idxv = lane + j * _L
                plsc.addupdate_scatter(hist, [lane_base + (idxv >> 5)],
                                       ones_i32, mask=m)
            return 0

        lax.fori_loop(0, _NV // 8, tie_a, 0)
        buck_a, below_a = _pick_bucket(hist, tie_budget)

        def tie_b(i, _):
            for u in range(8):
                j = i * 8 + u
                v = keys[pl.ds(j * _L, _L)]
                idxv = lane + j * _L
                m = (v == t_key) & ((idxv >> 5) == buck_a)
                plsc.addupdate_scatter(hist, [lane_base + (idxv & 31)],
                                       ones_i32, mask=m)
            return 0

        lax.fori_loop(0, _NV // 8, tie_b, 0)
        buck_b, _unused = _pick_bucket(hist, tie_budget - below_a)
        i_cut = buck_a * 32 + buck_b

        it_key = plsc.bitcast(
            jnp.full((_L,), t_key ^ jnp.uint32(0x80000000)), jnp.int32)

        def mask_body(i, _):
            for u in range(8):
                j = i * 8 + u
                v = keys[pl.ds(j * _L, _L)]
                ik = plsc.bitcast(v ^ jnp.uint32(0x80000000), jnp.int32)
                eq = v == t_key
                idxv = lane + j * _L
                zero = (ik < it_key) | (eq & (idxv <= i_cut))
                outv[pl.ds(j * _L, _L)] = jnp.where(zero, 0.0, 1.0)
            return 0

        lax.fori_loop(0, _NV // 8, mask_body, 0)
        pltpu.sync_copy(outv, o_hbm.at[wid])


@jax.jit
def _run_sc(x):
    f = pl.kernel(
        _sc_body,
        out_type=jax.ShapeDtypeStruct((3, _N), jnp.float32),
        mesh=plsc.VectorSubcoreMesh(core_axis_name="c", subcore_axis_name="s"),
        scratch_types=[
            pltpu.VMEM((_N,), jnp.float32),
            pltpu.VMEM((_N,), jnp.uint32),
            pltpu.VMEM((_NB * _L,), jnp.int32),
            pltpu.VMEM((_N,), jnp.float32),
        ],
        compiler_params=pltpu.CompilerParams(needs_layout_passes=False),
    )
    return f(x)


def kernel(self_att_layer_loga, cross_att_layer_loga, ffn_layer_loga):
    x = jnp.stack([self_att_layer_loga, cross_att_layer_loga, ffn_layer_loga])
    out = _run_sc(x)
    return (out[0], out[1], out[2])


# speedup vs baseline: 1.7440x; 1.0280x over previous
import functools

import numpy as np
import jax
import jax.numpy as jnp
from jax import lax
from jax.experimental import pallas as pl
from jax.experimental.pallas import tpu as pltpu
from jax.experimental.pallas import tpu_sc as plsc

_EPS = 1e-6
_LIMIT_A = -0.1
_LIMIT_B = 1.1
_BETA = 2.0 / 3.0
_XN = (0.0 - _LIMIT_A) / (_LIMIT_B - _LIMIT_A)
_C = float(np.log(_XN / (1.0 - _XN))) * _BETA

_N = 8192
_L = 16
_NV = _N // _L
_NB = 256
_BIG = np.int32(0x7FFFFFFF)
_STRIDE = _NB + 1
_HWORDS = -(-(_STRIDE * _L) // 128) * 128


def _round_half_even(x):
    tr = x.astype(jnp.int32)
    frac = x - tr.astype(jnp.float32)
    bump = jnp.where(frac > 0.5, jnp.int32(1),
                     jnp.where(frac == 0.5, tr & 1, jnp.int32(0)))
    return tr + bump


def _pick_bucket(hist, kk):
    lane = lax.iota(jnp.int32, _L)
    z16 = jnp.zeros((_L,), jnp.int32)

    def body(g, carry):
        run_min, cnt_carry = carry
        acc = z16
        for l in range(_L):
            sl = pl.ds(l * _STRIDE + g * _L, _L)
            acc = acc + hist[sl]
            hist[sl] = z16
        cum = plsc.cumsum(acc) + cnt_carry
        found = cum >= kk
        cand = jnp.where(found, ((g * _L + lane) << 14) | (cum - acc), _BIG)
        return jnp.minimum(run_min, cand), cnt_carry + jnp.sum(acc)

    run_min, _ = lax.fori_loop(0, _NB // _L, body,
                               (jnp.full((_L,), _BIG), jnp.int32(0)))
    cm = jnp.min(run_min)
    return cm >> 14, cm & jnp.int32(0x3FFF)


def _sc_body(x_hbm, o_hbm, xv, keys, hist, outv):
    wid = lax.axis_index("s") * 2 + lax.axis_index("c")
    lane = lax.iota(jnp.int32, _L)
    lane_base = lane * _STRIDE
    ones_i32 = jnp.ones((_L,), jnp.int32)
    z16_i32 = jnp.zeros((_L,), jnp.int32)

    @pl.when(wid < 3)
    def _():
        pltpu.sync_copy(x_hbm.at[wid], xv)

        def zero_body(i, _):
            for u in range(8):
                hist[pl.ds((i * 8 + u) * _L, _L)] = z16_i32
            return 0

        lax.fori_loop(0, _HWORDS // (8 * _L), zero_body, 0)

        def pass_a(i, accs):
            new = []
            for u in range(4):
                j = i * 4 + u
                v = xv[pl.ds(j * _L, _L)]
                e = jnp.exp(v - _C)
                s = 1.0 / (1.0 + e)
                s = jnp.clip(s, _EPS, 1.0 - _EPS)
                new.append(accs[u] + (1.0 - s))
                vc = jnp.where(v == 0.0, 0.0, v)
                b = plsc.bitcast(vc, jnp.uint32)
                flip = jnp.where((b >> 31) != 0,
                                 jnp.uint32(0xFFFFFFFF), jnp.uint32(0x80000000))
                uk = b ^ flip
                keys[pl.ds(j * _L, _L)] = uk
                pos = lane_base + (uk >> 24).astype(jnp.int32)
                plsc.addupdate_scatter(hist, [pos], ones_i32)
            return tuple(new)

        z16f = jnp.zeros((_L,), jnp.float32)
        a0, a1, a2, a3 = lax.fori_loop(0, _NV // 4, pass_a,
                                       (z16f, z16f, z16f, z16f))
        total = jnp.sum((a0 + a1) + (a2 + a3))
        k = _round_half_even(np.float32(_N) - total)

        bucket, below = _pick_bucket(hist, k)
        count_lt = below
        prefix = bucket.astype(jnp.uint32)
        for lvl in range(1, 4):
            shift = jnp.uint32(24 - 8 * lvl)

            def scan_body(i, _, shift=shift, prefix=prefix):
                for u in range(8):
                    j = i * 8 + u
                    v = keys[pl.ds(j * _L, _L)]
                    b = ((v >> shift) & jnp.uint32(0xFF)).astype(jnp.int32)
                    m = (v >> (shift + jnp.uint32(8))) == prefix
                    plsc.addupdate_scatter(hist, [lane_base + b], ones_i32,
                                           mask=m)
                return 0

            lax.fori_loop(0, _NV // 8, scan_body, 0)
            bucket, below = _pick_bucket(hist, k - count_lt)
            count_lt = count_lt + below
            prefix = (prefix << jnp.uint32(8)) | bucket.astype(jnp.uint32)
        t_key = prefix
        tie_budget = k - count_lt

        def tie_a(i, _):
            for u in range(8):
                j = i * 8 + u
                v = keys[pl.ds(j * _L, _L)]
                m = v == t_key
                idxv = lane + j * _L
                plsc.addupdate_scatter(hist, [lane_base + (idxv >> 5)],
                                       ones_i32, mask=m)
            return 0

        lax.fori_loop(0, _NV // 8, tie_a, 0)
        buck_a, below_a = _pick_bucket(hist, tie_budget)

        def tie_b(i, _):
            for u in range(8):
                j = i * 8 + u
                v = keys[pl.ds(j * _L, _L)]
                idxv = lane + j * _L
                m = (v == t_key) & ((idxv >> 5) == buck_a)
                plsc.addupdate_scatter(hist, [lane_base + (idxv & 31)],
                                       ones_i32, mask=m)
            return 0

        lax.fori_loop(0, _NV // 8, tie_b, 0)
        buck_b, _unused = _pick_bucket(hist, tie_budget - below_a)
        i_cut = buck_a * 32 + buck_b

        it_key = plsc.bitcast(
            jnp.full((_L,), t_key ^ jnp.uint32(0x80000000)), jnp.int32)

        def mask_body(i, _):
            for u in range(8):
                j = i * 8 + u
                v = keys[pl.ds(j * _L, _L)]
                ik = plsc.bitcast(v ^ jnp.uint32(0x80000000), jnp.int32)
                eq = v == t_key
                idxv = lane + j * _L
                zero = (ik < it_key) | (eq & (idxv <= i_cut))
                outv[pl.ds(j * _L, _L)] = jnp.where(zero, 0.0, 1.0)
            return 0

        lax.fori_loop(0, _NV // 8, mask_body, 0)
        pltpu.sync_copy(outv, o_hbm.at[wid])


@jax.jit
def _run_sc(x):
    f = pl.kernel(
        _sc_body,
        out_type=jax.ShapeDtypeStruct((3, _N), jnp.float32),
        mesh=plsc.VectorSubcoreMesh(core_axis_name="c", subcore_axis_name="s"),
        scratch_types=[
            pltpu.VMEM((_N,), jnp.float32),
            pltpu.VMEM((_N,), jnp.uint32),
            pltpu.VMEM((_HWORDS,), jnp.int32),
            pltpu.VMEM((_N,), jnp.float32),
        ],
        compiler_params=pltpu.CompilerParams(needs_layout_passes=False),
    )
    return f(x)


def kernel(self_att_layer_loga, cross_att_layer_loga, ffn_layer_loga):
    x = jnp.stack([self_att_layer_loga, cross_att_layer_loga, ffn_layer_loga])
    out = _run_sc(x)
    return (out[0], out[1], out[2])


# speedup vs baseline: 2.7631x; 1.5843x over previous
import functools

import numpy as np
import jax
import jax.numpy as jnp
from jax import lax
from jax.experimental import pallas as pl
from jax.experimental.pallas import tpu as pltpu
from jax.experimental.pallas import tpu_sc as plsc

_EPS = 1e-6
_LIMIT_A = -0.1
_LIMIT_B = 1.1
_BETA = 2.0 / 3.0
_XN = (0.0 - _LIMIT_A) / (_LIMIT_B - _LIMIT_A)
_C = float(np.log(_XN / (1.0 - _XN))) * _BETA

_N = 8192
_L = 16
_NV = _N // _L
_NB = 256
_BIG = np.int32(0x7FFFFFFF)
_STRIDE = _NB + 1
_HWORDS = -(-(_STRIDE * _L) // 128) * 128


def _round_half_even(x):
    tr = x.astype(jnp.int32)
    frac = x - tr.astype(jnp.float32)
    bump = jnp.where(frac > 0.5, jnp.int32(1),
                     jnp.where(frac == 0.5, tr & 1, jnp.int32(0)))
    return tr + bump


def _pick_bucket(hist, kk):
    lane = lax.iota(jnp.int32, _L)
    z16 = jnp.zeros((_L,), jnp.int32)

    def body(g, carry):
        run_min, cnt_carry = carry
        acc = z16
        for l in range(_L):
            sl = pl.ds(l * _STRIDE + g * _L, _L)
            acc = acc + hist[sl]
            hist[sl] = z16
        cum = plsc.cumsum(acc) + cnt_carry
        found = cum >= kk
        cand = jnp.where(found, ((g * _L + lane) << 14) | (cum - acc), _BIG)
        return jnp.minimum(run_min, cand), cnt_carry + jnp.sum(acc)

    run_min, _ = lax.fori_loop(0, _NB // _L, body,
                               (jnp.full((_L,), _BIG), jnp.int32(0)))
    cm = jnp.min(run_min)
    return cm >> 14, cm & jnp.int32(0x3FFF)


def _sc_body(x_hbm, o_hbm, xv, keys, hist, outv):
    wid = lax.axis_index("s") * 2 + lax.axis_index("c")
    lane = lax.iota(jnp.int32, _L)
    lane_base = lane * _STRIDE
    ones_i32 = jnp.ones((_L,), jnp.int32)
    z16_i32 = jnp.zeros((_L,), jnp.int32)

    @pl.when(wid < 3)
    def _():
        pltpu.sync_copy(x_hbm.at[wid], xv)

        def zero_body(i, _):
            for u in range(8):
                hist[pl.ds((i * 8 + u) * _L, _L)] = z16_i32
            return 0

        lax.fori_loop(0, _HWORDS // (8 * _L), zero_body, 0)

        def pass_a(i, accs):
            vs = [xv[pl.ds((i * 4 + u) * _L, _L)] for u in range(4)]
            new, uks, poss = [], [], []
            for u in range(4):
                v = vs[u]
                e = jnp.exp(v - _C)
                s = 1.0 / (1.0 + e)
                s = jnp.clip(s, _EPS, 1.0 - _EPS)
                new.append(accs[u] + (1.0 - s))
                vc = jnp.where(v == 0.0, 0.0, v)
                b = plsc.bitcast(vc, jnp.uint32)
                flip = jnp.where((b >> 31) != 0,
                                 jnp.uint32(0xFFFFFFFF), jnp.uint32(0x80000000))
                uk = b ^ flip
                uks.append(uk)
                poss.append(lane_base + (uk >> 24).astype(jnp.int32))
            for u in range(4):
                keys[pl.ds((i * 4 + u) * _L, _L)] = uks[u]
            for u in range(4):
                plsc.addupdate_scatter(hist, [poss[u]], ones_i32)
            return tuple(new)

        z16f = jnp.zeros((_L,), jnp.float32)
        a0, a1, a2, a3 = lax.fori_loop(0, _NV // 4, pass_a,
                                       (z16f, z16f, z16f, z16f))
        total = jnp.sum((a0 + a1) + (a2 + a3))
        k = _round_half_even(np.float32(_N) - total)

        bucket, below = _pick_bucket(hist, k)
        count_lt = below
        prefix = bucket.astype(jnp.uint32)
        for lvl in range(1, 4):
            shift = jnp.uint32(24 - 8 * lvl)

            def scan_body(i, _, shift=shift, prefix=prefix):
                vs = [keys[pl.ds((i * 8 + u) * _L, _L)] for u in range(8)]
                poss, ms = [], []
                for u in range(8):
                    v = vs[u]
                    b = ((v >> shift) & jnp.uint32(0xFF)).astype(jnp.int32)
                    poss.append(lane_base + b)
                    ms.append((v >> (shift + jnp.uint32(8))) == prefix)
                for u in range(8):
                    plsc.addupdate_scatter(hist, [poss[u]], ones_i32,
                                           mask=ms[u])
                return 0

            lax.fori_loop(0, _NV // 8, scan_body, 0)
            bucket, below = _pick_bucket(hist, k - count_lt)
            count_lt = count_lt + below
            prefix = (prefix << jnp.uint32(8)) | bucket.astype(jnp.uint32)
        t_key = prefix
        tie_budget = k - count_lt

        def tie_a(i, _):
            vs = [keys[pl.ds((i * 8 + u) * _L, _L)] for u in range(8)]
            poss, ms = [], []
            for u in range(8):
                idxv = lane + (i * 8 + u) * _L
                poss.append(lane_base + (idxv >> 5))
                ms.append(vs[u] == t_key)
            for u in range(8):
                plsc.addupdate_scatter(hist, [poss[u]], ones_i32, mask=ms[u])
            return 0

        lax.fori_loop(0, _NV // 8, tie_a, 0)
        buck_a, below_a = _pick_bucket(hist, tie_budget)

        def tie_b(i, _):
            vs = [keys[pl.ds((i * 8 + u) * _L, _L)] for u in range(8)]
            poss, ms = [], []
            for u in range(8):
                idxv = lane + (i * 8 + u) * _L
                poss.append(lane_base + (idxv & 31))
                ms.append((vs[u] == t_key) & ((idxv >> 5) == buck_a))
            for u in range(8):
                plsc.addupdate_scatter(hist, [poss[u]], ones_i32, mask=ms[u])
            return 0

        lax.fori_loop(0, _NV // 8, tie_b, 0)
        buck_b, _unused = _pick_bucket(hist, tie_budget - below_a)
        i_cut = buck_a * 32 + buck_b

        it_key = plsc.bitcast(
            jnp.full((_L,), t_key ^ jnp.uint32(0x80000000)), jnp.int32)

        def mask_body(i, _):
            vs = [keys[pl.ds((i * 8 + u) * _L, _L)] for u in range(8)]
            outs = []
            for u in range(8):
                v = vs[u]
                ik = plsc.bitcast(v ^ jnp.uint32(0x80000000), jnp.int32)
                eq = v == t_key
                idxv = lane + (i * 8 + u) * _L
                zero = (ik < it_key) | (eq & (idxv <= i_cut))
                outs.append(jnp.where(zero, 0.0, 1.0))
            for u in range(8):
                outv[pl.ds((i * 8 + u) * _L, _L)] = outs[u]
            return 0

        lax.fori_loop(0, _NV // 8, mask_body, 0)
        pltpu.sync_copy(outv, o_hbm.at[wid])


@jax.jit
def _run_sc(x):
    f = pl.kernel(
        _sc_body,
        out_type=jax.ShapeDtypeStruct((3, _N), jnp.float32),
        mesh=plsc.VectorSubcoreMesh(core_axis_name="c", subcore_axis_name="s"),
        scratch_types=[
            pltpu.VMEM((_N,), jnp.float32),
            pltpu.VMEM((_N,), jnp.uint32),
            pltpu.VMEM((_HWORDS,), jnp.int32),
            pltpu.VMEM((_N,), jnp.float32),
        ],
        compiler_params=pltpu.CompilerParams(needs_layout_passes=False),
    )
    return f(x)


def kernel(self_att_layer_loga, cross_att_layer_loga, ffn_layer_loga):
    x = jnp.stack([self_att_layer_loga, cross_att_layer_loga, ffn_layer_loga])
    out = _run_sc(x)
    return (out[0], out[1], out[2])


# speedup vs baseline: 2.8855x; 1.0443x over previous
import functools

import numpy as np
import jax
import jax.numpy as jnp
from jax import lax
from jax.experimental import pallas as pl
from jax.experimental.pallas import tpu as pltpu
from jax.experimental.pallas import tpu_sc as plsc

_EPS = 1e-6
_LIMIT_A = -0.1
_LIMIT_B = 1.1
_BETA = 2.0 / 3.0
_XN = (0.0 - _LIMIT_A) / (_LIMIT_B - _LIMIT_A)
_C = float(np.log(_XN / (1.0 - _XN))) * _BETA

_N = 8192
_L = 16
_NV = _N // _L
_NB = 256
_BIG = np.int32(0x7FFFFFFF)
_STRIDE = _NB + 1
_HWORDS = -(-(_STRIDE * _L) // 128) * 128


def _round_half_even(x):
    tr = x.astype(jnp.int32)
    frac = x - tr.astype(jnp.float32)
    bump = jnp.where(frac > 0.5, jnp.int32(1),
                     jnp.where(frac == 0.5, tr & 1, jnp.int32(0)))
    return tr + bump


def _pick_bucket(hist, kk):
    lane = lax.iota(jnp.int32, _L)
    z16 = jnp.zeros((_L,), jnp.int32)

    def body(g, carry):
        run_min, cnt_carry = carry
        acc = z16
        for l in range(_L):
            sl = pl.ds(l * _STRIDE + g * _L, _L)
            acc = acc + hist[sl]
            hist[sl] = z16
        cum = plsc.cumsum(acc) + cnt_carry
        found = cum >= kk
        cand = jnp.where(found, ((g * _L + lane) << 14) | (cum - acc), _BIG)
        return jnp.minimum(run_min, cand), cnt_carry + jnp.sum(acc)

    run_min, _ = lax.fori_loop(0, _NB // _L, body,
                               (jnp.full((_L,), _BIG), jnp.int32(0)))
    cm = jnp.min(run_min)
    return cm >> 14, cm & jnp.int32(0x3FFF)


def _sc_body(x0_hbm, x1_hbm, x2_hbm, o0_hbm, o1_hbm, o2_hbm,
             xv, keys, hist, outv):
    wid = lax.axis_index("s") * 2 + lax.axis_index("c")
    lane = lax.iota(jnp.int32, _L)
    lane_base = lane * _STRIDE
    ones_i32 = jnp.ones((_L,), jnp.int32)
    z16_i32 = jnp.zeros((_L,), jnp.int32)
    ins = (x0_hbm, x1_hbm, x2_hbm)
    outs = (o0_hbm, o1_hbm, o2_hbm)

    @pl.when(wid < 3)
    def _():
        for w in range(3):
            @pl.when(wid == w)
            def _(w=w):
                pltpu.sync_copy(ins[w], xv)

        def zero_body(i, _):
            for u in range(8):
                hist[pl.ds((i * 8 + u) * _L, _L)] = z16_i32
            return 0

        lax.fori_loop(0, _HWORDS // (8 * _L), zero_body, 0)

        def pass_a(i, accs):
            vs = [xv[pl.ds((i * 4 + u) * _L, _L)] for u in range(4)]
            new, uks, poss = [], [], []
            for u in range(4):
                v = vs[u]
                e = jnp.exp(v - _C)
                s = 1.0 / (1.0 + e)
                s = jnp.clip(s, _EPS, 1.0 - _EPS)
                new.append(accs[u] + (1.0 - s))
                vc = jnp.where(v == 0.0, 0.0, v)
                b = plsc.bitcast(vc, jnp.uint32)
                flip = jnp.where((b >> 31) != 0,
                                 jnp.uint32(0xFFFFFFFF), jnp.uint32(0x80000000))
                uk = b ^ flip
                uks.append(uk)
                poss.append(lane_base + (uk >> 24).astype(jnp.int32))
            for u in range(4):
                keys[pl.ds((i * 4 + u) * _L, _L)] = uks[u]
            for u in range(4):
                plsc.addupdate_scatter(hist, [poss[u]], ones_i32)
            return tuple(new)

        z16f = jnp.zeros((_L,), jnp.float32)
        a0, a1, a2, a3 = lax.fori_loop(0, _NV // 4, pass_a,
                                       (z16f, z16f, z16f, z16f))
        total = jnp.sum((a0 + a1) + (a2 + a3))
        k = _round_half_even(np.float32(_N) - total)

        bucket, below = _pick_bucket(hist, k)
        count_lt = below
        prefix = bucket.astype(jnp.uint32)
        for lvl in range(1, 4):
            shift = jnp.uint32(24 - 8 * lvl)

            def scan_body(i, _, shift=shift, prefix=prefix):
                vs = [keys[pl.ds((i * 8 + u) * _L, _L)] for u in range(8)]
                poss, ms = [], []
                for u in range(8):
                    v = vs[u]
                    b = ((v >> shift) & jnp.uint32(0xFF)).astype(jnp.int32)
                    poss.append(lane_base + b)
                    ms.append((v >> (shift + jnp.uint32(8))) == prefix)
                for u in range(8):
                    plsc.addupdate_scatter(hist, [poss[u]], ones_i32,
                                           mask=ms[u])
                return 0

            lax.fori_loop(0, _NV // 8, scan_body, 0)
            bucket, below = _pick_bucket(hist, k - count_lt)
            count_lt = count_lt + below
            prefix = (prefix << jnp.uint32(8)) | bucket.astype(jnp.uint32)
        t_key = prefix
        tie_budget = k - count_lt

        def tie_a(i, _):
            vs = [keys[pl.ds((i * 8 + u) * _L, _L)] for u in range(8)]
            poss, ms = [], []
            for u in range(8):
                idxv = lane + (i * 8 + u) * _L
                poss.append(lane_base + (idxv >> 5))
                ms.append(vs[u] == t_key)
            for u in range(8):
                plsc.addupdate_scatter(hist, [poss[u]], ones_i32, mask=ms[u])
            return 0

        lax.fori_loop(0, _NV // 8, tie_a, 0)
        buck_a, below_a = _pick_bucket(hist, tie_budget)

        def tie_b(i, _):
            vs = [keys[pl.ds((i * 8 + u) * _L, _L)] for u in range(8)]
            poss, ms = [], []
            for u in range(8):
                idxv = lane + (i * 8 + u) * _L
                poss.append(lane_base + (idxv & 31))
                ms.append((vs[u] == t_key) & ((idxv >> 5) == buck_a))
            for u in range(8):
                plsc.addupdate_scatter(hist, [poss[u]], ones_i32, mask=ms[u])
            return 0

        lax.fori_loop(0, _NV // 8, tie_b, 0)
        buck_b, _unused = _pick_bucket(hist, tie_budget - below_a)
        i_cut = buck_a * 32 + buck_b

        it_key = plsc.bitcast(
            jnp.full((_L,), t_key ^ jnp.uint32(0x80000000)), jnp.int32)

        def mask_body(i, _):
            vs = [keys[pl.ds((i * 8 + u) * _L, _L)] for u in range(8)]
            outs = []
            for u in range(8):
                v = vs[u]
                ik = plsc.bitcast(v ^ jnp.uint32(0x80000000), jnp.int32)
                eq = v == t_key
                idxv = lane + (i * 8 + u) * _L
                zero = (ik < it_key) | (eq & (idxv <= i_cut))
                outs.append(jnp.where(zero, 0.0, 1.0))
            for u in range(8):
                outv[pl.ds((i * 8 + u) * _L, _L)] = outs[u]
            return 0

        lax.fori_loop(0, _NV // 8, mask_body, 0)
        for w in range(3):
            @pl.when(wid == w)
            def _(w=w):
                pltpu.sync_copy(outv, outs[w])


@jax.jit
def _run_sc(x0, x1, x2):
    f = pl.kernel(
        _sc_body,
        out_type=(jax.ShapeDtypeStruct((_N,), jnp.float32),) * 3,
        mesh=plsc.VectorSubcoreMesh(core_axis_name="c", subcore_axis_name="s"),
        scratch_types=[
            pltpu.VMEM((_N,), jnp.float32),
            pltpu.VMEM((_N,), jnp.uint32),
            pltpu.VMEM((_HWORDS,), jnp.int32),
            pltpu.VMEM((_N,), jnp.float32),
        ],
        compiler_params=pltpu.CompilerParams(needs_layout_passes=False),
    )
    return f(x0, x1, x2)


def kernel(self_att_layer_loga, cross_att_layer_loga, ffn_layer_loga):
    return _run_sc(self_att_layer_loga, cross_att_layer_loga, ffn_layer_loga)


# speedup vs baseline: 3.0251x; 1.0484x over previous
import functools

import numpy as np
import jax
import jax.numpy as jnp
from jax import lax
from jax.experimental import pallas as pl
from jax.experimental.pallas import tpu as pltpu
from jax.experimental.pallas import tpu_sc as plsc

_EPS = 1e-6
_LIMIT_A = -0.1
_LIMIT_B = 1.1
_BETA = 2.0 / 3.0
_XN = (0.0 - _LIMIT_A) / (_LIMIT_B - _LIMIT_A)
_C = float(np.log(_XN / (1.0 - _XN))) * _BETA

_N = 8192
_L = 16
_NV = _N // _L
_NB = 256
_BIG = np.int32(0x7FFFFFFF)
_STRIDE = _NB + 1
_HWORDS = -(-(_STRIDE * _L) // 128) * 128


def _round_half_even(x):
    tr = x.astype(jnp.int32)
    frac = x - tr.astype(jnp.float32)
    bump = jnp.where(frac > 0.5, jnp.int32(1),
                     jnp.where(frac == 0.5, tr & 1, jnp.int32(0)))
    return tr + bump


def _pick_bucket(hist, kk):
    lane = lax.iota(jnp.int32, _L)
    z16 = jnp.zeros((_L,), jnp.int32)

    def body(g, carry):
        run_min, run_min2, cnt_carry = carry
        acc = z16
        for l in range(_L):
            sl = pl.ds(l * _STRIDE + g * _L, _L)
            acc = acc + hist[sl]
            hist[sl] = z16
        cum = plsc.cumsum(acc) + cnt_carry
        found = cum >= kk
        base = (g * _L + lane) << 14
        cand = jnp.where(found, base | (cum - acc), _BIG)
        cand2 = jnp.where(found, base | cum, _BIG)
        return (jnp.minimum(run_min, cand), jnp.minimum(run_min2, cand2),
                cnt_carry + jnp.sum(acc))

    run_min, run_min2, _ = lax.fori_loop(
        0, _NB // _L, body,
        (jnp.full((_L,), _BIG), jnp.full((_L,), _BIG), jnp.int32(0)))
    cm = jnp.min(run_min)
    below = cm & jnp.int32(0x3FFF)
    in_bucket = (jnp.min(run_min2) & jnp.int32(0x3FFF)) - below
    return cm >> 14, below, in_bucket


def _sc_body(x0_hbm, x1_hbm, x2_hbm, o0_hbm, o1_hbm, o2_hbm,
             xv, keys, hist, outv):
    wid = lax.axis_index("s") * 2 + lax.axis_index("c")
    lane = lax.iota(jnp.int32, _L)
    lane_base = lane * _STRIDE
    ones_i32 = jnp.ones((_L,), jnp.int32)
    z16_i32 = jnp.zeros((_L,), jnp.int32)
    ins = (x0_hbm, x1_hbm, x2_hbm)
    outs = (o0_hbm, o1_hbm, o2_hbm)

    @pl.when(wid < 3)
    def _():
        for w in range(3):
            @pl.when(wid == w)
            def _(w=w):
                pltpu.sync_copy(ins[w], xv)

        def zero_body(i, _):
            for u in range(8):
                hist[pl.ds((i * 8 + u) * _L, _L)] = z16_i32
            return 0

        lax.fori_loop(0, _HWORDS // (8 * _L), zero_body, 0)

        def pass_a(i, accs):
            vs = [xv[pl.ds((i * 4 + u) * _L, _L)] for u in range(4)]
            new, uks, poss = [], [], []
            for u in range(4):
                v = vs[u]
                e = jnp.exp(v - _C)
                s = 1.0 / (1.0 + e)
                s = jnp.clip(s, _EPS, 1.0 - _EPS)
                new.append(accs[u] + (1.0 - s))
                vc = jnp.where(v == 0.0, 0.0, v)
                b = plsc.bitcast(vc, jnp.uint32)
                flip = jnp.where((b >> 31) != 0,
                                 jnp.uint32(0xFFFFFFFF), jnp.uint32(0x80000000))
                uk = b ^ flip
                uks.append(uk)
                poss.append(lane_base + (uk >> 24).astype(jnp.int32))
            for u in range(4):
                keys[pl.ds((i * 4 + u) * _L, _L)] = uks[u]
            for u in range(4):
                plsc.addupdate_scatter(hist, [poss[u]], ones_i32)
            return tuple(new)

        z16f = jnp.zeros((_L,), jnp.float32)
        a0, a1, a2, a3 = lax.fori_loop(0, _NV // 4, pass_a,
                                       (z16f, z16f, z16f, z16f))
        total = jnp.sum((a0 + a1) + (a2 + a3))
        k = _round_half_even(np.float32(_N) - total)

        bucket, below, _cnt = _pick_bucket(hist, k)
        count_lt = below
        prefix = bucket.astype(jnp.uint32)
        for lvl in range(1, 4):
            shift = jnp.uint32(24 - 8 * lvl)

            def scan_body(i, _, shift=shift, prefix=prefix):
                vs = [keys[pl.ds((i * 8 + u) * _L, _L)] for u in range(8)]
                poss, ms = [], []
                for u in range(8):
                    v = vs[u]
                    b = ((v >> shift) & jnp.uint32(0xFF)).astype(jnp.int32)
                    poss.append(lane_base + b)
                    ms.append((v >> (shift + jnp.uint32(8))) == prefix)
                for u in range(8):
                    plsc.addupdate_scatter(hist, [poss[u]], ones_i32,
                                           mask=ms[u])
                return 0

            lax.fori_loop(0, _NV // 8, scan_body, 0)
            bucket, below, cnt_eq = _pick_bucket(hist, k - count_lt)
            count_lt = count_lt + below
            prefix = (prefix << jnp.uint32(8)) | bucket.astype(jnp.uint32)
        t_key = prefix
        tie_budget = k - count_lt
        need_tie = cnt_eq > tie_budget
        tie_trips = jnp.where(need_tie, _NV // 8, 0)

        def tie_a(i, _):
            vs = [keys[pl.ds((i * 8 + u) * _L, _L)] for u in range(8)]
            poss, ms = [], []
            for u in range(8):
                idxv = lane + (i * 8 + u) * _L
                poss.append(lane_base + (idxv >> 5))
                ms.append(vs[u] == t_key)
            for u in range(8):
                plsc.addupdate_scatter(hist, [poss[u]], ones_i32, mask=ms[u])
            return 0

        lax.fori_loop(0, tie_trips, tie_a, 0)
        buck_a, below_a, _ca = _pick_bucket(hist, tie_budget)

        def tie_b(i, _):
            vs = [keys[pl.ds((i * 8 + u) * _L, _L)] for u in range(8)]
            poss, ms = [], []
            for u in range(8):
                idxv = lane + (i * 8 + u) * _L
                poss.append(lane_base + (idxv & 31))
                ms.append((vs[u] == t_key) & ((idxv >> 5) == buck_a))
            for u in range(8):
                plsc.addupdate_scatter(hist, [poss[u]], ones_i32, mask=ms[u])
            return 0

        lax.fori_loop(0, tie_trips, tie_b, 0)
        buck_b, _ub, _cb = _pick_bucket(hist, tie_budget - below_a)
        i_cut = jnp.where(need_tie, buck_a * 32 + buck_b, jnp.int32(_N))

        it_key = plsc.bitcast(
            jnp.full((_L,), t_key ^ jnp.uint32(0x80000000)), jnp.int32)

        def mask_body(i, _):
            vs = [keys[pl.ds((i * 8 + u) * _L, _L)] for u in range(8)]
            outs = []
            for u in range(8):
                v = vs[u]
                ik = plsc.bitcast(v ^ jnp.uint32(0x80000000), jnp.int32)
                eq = v == t_key
                idxv = lane + (i * 8 + u) * _L
                zero = (ik < it_key) | (eq & (idxv <= i_cut))
                outs.append(jnp.where(zero, 0.0, 1.0))
            for u in range(8):
                outv[pl.ds((i * 8 + u) * _L, _L)] = outs[u]
            return 0

        lax.fori_loop(0, _NV // 8, mask_body, 0)
        for w in range(3):
            @pl.when(wid == w)
            def _(w=w):
                pltpu.sync_copy(outv, outs[w])


@jax.jit
def _run_sc(x0, x1, x2):
    f = pl.kernel(
        _sc_body,
        out_type=(jax.ShapeDtypeStruct((_N,), jnp.float32),) * 3,
        mesh=plsc.VectorSubcoreMesh(core_axis_name="c", subcore_axis_name="s"),
        scratch_types=[
            pltpu.VMEM((_N,), jnp.float32),
            pltpu.VMEM((_N,), jnp.uint32),
            pltpu.VMEM((_HWORDS,), jnp.int32),
            pltpu.VMEM((_N,), jnp.float32),
        ],
        compiler_params=pltpu.CompilerParams(needs_layout_passes=False),
    )
    return f(x0, x1, x2)


def kernel(self_att_layer_loga, cross_att_layer_loga, ffn_layer_loga):
    return _run_sc(self_att_layer_loga, cross_att_layer_loga, ffn_layer_loga)


# speedup vs baseline: 3.0718x; 1.0154x over previous
import functools

import numpy as np
import jax
import jax.numpy as jnp
from jax import lax
from jax.experimental import pallas as pl
from jax.experimental.pallas import tpu as pltpu
from jax.experimental.pallas import tpu_sc as plsc

_EPS = 1e-6
_LIMIT_A = -0.1
_LIMIT_B = 1.1
_BETA = 2.0 / 3.0
_XN = (0.0 - _LIMIT_A) / (_LIMIT_B - _LIMIT_A)
_C = float(np.log(_XN / (1.0 - _XN))) * _BETA

_N = 8192
_L = 16
_NV = _N // _L
_NB = 256
_BIG = np.int32(0x7FFFFFFF)
_STRIDE = _NB + 1
_HWORDS = -(-(_STRIDE * _L) // 128) * 128


def _round_half_even(x):
    tr = x.astype(jnp.int32)
    frac = x - tr.astype(jnp.float32)
    bump = jnp.where(frac > 0.5, jnp.int32(1),
                     jnp.where(frac == 0.5, tr & 1, jnp.int32(0)))
    return tr + bump


def _pick_bucket(hist, kk):
    lane = lax.iota(jnp.int32, _L)
    z16 = jnp.zeros((_L,), jnp.int32)

    def body(g, carry):
        run_min, run_min2, cnt_carry = carry
        acc = z16
        for l in range(_L):
            sl = pl.ds(l * _STRIDE + g * _L, _L)
            acc = acc + hist[sl]
            hist[sl] = z16
        cum = plsc.cumsum(acc) + cnt_carry
        found = cum >= kk
        base = (g * _L + lane) << 14
        cand = jnp.where(found, base | (cum - acc), _BIG)
        cand2 = jnp.where(found, base | cum, _BIG)
        return (jnp.minimum(run_min, cand), jnp.minimum(run_min2, cand2),
                cnt_carry + jnp.sum(acc))

    run_min, run_min2, _ = lax.fori_loop(
        0, _NB // _L, body,
        (jnp.full((_L,), _BIG), jnp.full((_L,), _BIG), jnp.int32(0)))
    cm = jnp.min(run_min)
    below = cm & jnp.int32(0x3FFF)
    in_bucket = (jnp.min(run_min2) & jnp.int32(0x3FFF)) - below
    return cm >> 14, below, in_bucket


def _sc_body(x0_hbm, x1_hbm, x2_hbm, o0_hbm, o1_hbm, o2_hbm,
             xv, keys, hist, outv):
    wid = lax.axis_index("s") * 2 + lax.axis_index("c")
    lane = lax.iota(jnp.int32, _L)
    lane_base = lane * _STRIDE
    ones_i32 = jnp.ones((_L,), jnp.int32)
    z16_i32 = jnp.zeros((_L,), jnp.int32)
    ins = (x0_hbm, x1_hbm, x2_hbm)
    outs = (o0_hbm, o1_hbm, o2_hbm)

    @pl.when(wid < 3)
    def _():
        for w in range(3):
            @pl.when(wid == w)
            def _(w=w):
                pltpu.sync_copy(ins[w], xv)

        def zero_body(i, _):
            for u in range(8):
                hist[pl.ds((i * 8 + u) * _L, _L)] = z16_i32
            return 0

        lax.fori_loop(0, _HWORDS // (8 * _L), zero_body, 0)

        def pass_a(i, accs):
            vs = [xv[pl.ds((i * 4 + u) * _L, _L)] for u in range(4)]
            new, uks, poss = [], [], []
            for u in range(4):
                v = vs[u]
                e = jnp.exp(v - _C)
                s = 1.0 / (1.0 + e)
                s = jnp.clip(s, _EPS, 1.0 - _EPS)
                new.append(accs[u] + (1.0 - s))
                vc = jnp.where(v == 0.0, 0.0, v)
                b = plsc.bitcast(vc, jnp.uint32)
                flip = jnp.where((b >> 31) != 0,
                                 jnp.uint32(0xFFFFFFFF), jnp.uint32(0x80000000))
                uk = b ^ flip
                uks.append(uk)
                poss.append(lane_base + (uk >> 24).astype(jnp.int32))
            for u in range(4):
                keys[pl.ds((i * 4 + u) * _L, _L)] = uks[u]
            for u in range(4):
                plsc.addupdate_scatter(hist, [poss[u]], ones_i32)
            return tuple(new)

        z16f = jnp.zeros((_L,), jnp.float32)
        a0, a1, a2, a3 = lax.fori_loop(0, _NV // 4, pass_a,
                                       (z16f, z16f, z16f, z16f))
        total = jnp.sum((a0 + a1) + (a2 + a3))
        k = _round_half_even(np.float32(_N) - total)

        bucket, below, _cnt = _pick_bucket(hist, k)
        count_lt = below
        prefix = bucket.astype(jnp.uint32)
        for lvl in range(1, 4):
            shift = jnp.uint32(24 - 8 * lvl)

            def scan_body(i, _, shift=shift, prefix=prefix):
                vs = [keys[pl.ds((i * 8 + u) * _L, _L)] for u in range(8)]
                poss, ms = [], []
                for u in range(8):
                    v = vs[u]
                    b = ((v >> shift) & jnp.uint32(0xFF)).astype(jnp.int32)
                    poss.append(lane_base + b)
                    ms.append((v >> (shift + jnp.uint32(8))) == prefix)
                for u in range(8):
                    plsc.addupdate_scatter(hist, [poss[u]], ones_i32,
                                           mask=ms[u])
                return 0

            lax.fori_loop(0, _NV // 8, scan_body, 0)
            bucket, below, cnt_eq = _pick_bucket(hist, k - count_lt)
            count_lt = count_lt + below
            prefix = (prefix << jnp.uint32(8)) | bucket.astype(jnp.uint32)
        t_key = prefix
        tie_budget = k - count_lt
        need_tie = cnt_eq > tie_budget
        tie_trips = jnp.where(need_tie, _NV // 8, 0)

        def tie_a(i, _):
            vs = [keys[pl.ds((i * 8 + u) * _L, _L)] for u in range(8)]
            poss, ms = [], []
            for u in range(8):
                idxv = lane + (i * 8 + u) * _L
                poss.append(lane_base + (idxv >> 5))
                ms.append(vs[u] == t_key)
            for u in range(8):
                plsc.addupdate_scatter(hist, [poss[u]], ones_i32, mask=ms[u])
            return 0

        lax.fori_loop(0, tie_trips, tie_a, 0)
        buck_a, below_a, _ca = _pick_bucket(hist, tie_budget)

        def tie_b(i, _):
            vs = [keys[pl.ds((i * 8 + u) * _L, _L)] for u in range(8)]
            poss, ms = [], []
            for u in range(8):
                idxv = lane + (i * 8 + u) * _L
                poss.append(lane_base + (idxv & 31))
                ms.append((vs[u] == t_key) & ((idxv >> 5) == buck_a))
            for u in range(8):
                plsc.addupdate_scatter(hist, [poss[u]], ones_i32, mask=ms[u])
            return 0

        lax.fori_loop(0, tie_trips, tie_b, 0)
        buck_b, _ub, _cb = _pick_bucket(hist, tie_budget - below_a)
        i_cut = jnp.where(need_tie, buck_a * 32 + buck_b, jnp.int32(_N))

        it_key = plsc.bitcast(
            jnp.full((_L,), t_key ^ jnp.uint32(0x80000000)), jnp.int32)

        def mask_body(i, _):
            vs = [keys[pl.ds((i * 4 + u) * _L, _L)] for u in range(4)]
            outs = []
            for u in range(4):
                v = vs[u]
                ik = plsc.bitcast(v ^ jnp.uint32(0x80000000), jnp.int32)
                eq = v == t_key
                idxv = lane + (i * 4 + u) * _L
                zero = (ik < it_key) | (eq & (idxv <= i_cut))
                outs.append(jnp.where(zero, 0.0, 1.0))
            for u in range(4):
                outv[pl.ds((i * 4 + u) * _L, _L)] = outs[u]
            return 0

        lax.fori_loop(0, _NV // 4, mask_body, 0)
        for w in range(3):
            @pl.when(wid == w)
            def _(w=w):
                pltpu.sync_copy(outv, outs[w])


@jax.jit
def _run_sc(x0, x1, x2):
    f = pl.kernel(
        _sc_body,
        out_type=(jax.ShapeDtypeStruct((_N,), jnp.float32),) * 3,
        mesh=plsc.VectorSubcoreMesh(core_axis_name="c", subcore_axis_name="s"),
        scratch_types=[
            pltpu.VMEM((_N,), jnp.float32),
            pltpu.VMEM((_N,), jnp.uint32),
            pltpu.VMEM((_HWORDS,), jnp.int32),
            pltpu.VMEM((_N,), jnp.float32),
        ],
        compiler_params=pltpu.CompilerParams(needs_layout_passes=False),
    )
    return f(x0, x1, x2)


def kernel(self_att_layer_loga, cross_att_layer_loga, ffn_layer_loga):
    return _run_sc(self_att_layer_loga, cross_att_layer_loga, ffn_layer_loga)


# speedup vs baseline: 3.0773x; 1.0018x over previous
import functools

import numpy as np
import jax
import jax.numpy as jnp
from jax import lax
from jax.experimental import pallas as pl
from jax.experimental.pallas import tpu as pltpu
from jax.experimental.pallas import tpu_sc as plsc

_EPS = 1e-6
_LIMIT_A = -0.1
_LIMIT_B = 1.1
_BETA = 2.0 / 3.0
_XN = (0.0 - _LIMIT_A) / (_LIMIT_B - _LIMIT_A)
_C = float(np.log(_XN / (1.0 - _XN))) * _BETA

_N = 8192
_L = 16
_NV = _N // _L
_NB = 256
_BIG = np.int32(0x7FFFFFFF)
_STRIDE = _NB + 1
_HWORDS = -(-(_STRIDE * _L) // 128) * 128


def _round_half_even(x):
    tr = x.astype(jnp.int32)
    frac = x - tr.astype(jnp.float32)
    bump = jnp.where(frac > 0.5, jnp.int32(1),
                     jnp.where(frac == 0.5, tr & 1, jnp.int32(0)))
    return tr + bump


def _pick_bucket(hist, kk):
    lane = lax.iota(jnp.int32, _L)
    z16 = jnp.zeros((_L,), jnp.int32)

    def body(g, carry):
        run_min, run_min2, cnt_carry = carry
        acc = z16
        for l in range(_L):
            sl = pl.ds(l * _STRIDE + g * _L, _L)
            acc = acc + hist[sl]
            hist[sl] = z16
        cum = plsc.cumsum(acc) + cnt_carry
        found = cum >= kk
        base = (g * _L + lane) << 14
        cand = jnp.where(found, base | (cum - acc), _BIG)
        cand2 = jnp.where(found, base | cum, _BIG)
        return (jnp.minimum(run_min, cand), jnp.minimum(run_min2, cand2),
                cnt_carry + jnp.sum(acc))

    run_min, run_min2, _ = lax.fori_loop(
        0, _NB // _L, body,
        (jnp.full((_L,), _BIG), jnp.full((_L,), _BIG), jnp.int32(0)))
    cm = jnp.min(run_min)
    below = cm & jnp.int32(0x3FFF)
    in_bucket = (jnp.min(run_min2) & jnp.int32(0x3FFF)) - below
    return cm >> 14, below, in_bucket


def _sc_body(x0_hbm, x1_hbm, x2_hbm, o0_hbm, o1_hbm, o2_hbm,
             xv, keys, hist, outv):
    wid = lax.axis_index("s") * 2 + lax.axis_index("c")
    lane = lax.iota(jnp.int32, _L)
    lane_base = lane * _STRIDE
    ones_i32 = jnp.ones((_L,), jnp.int32)
    z16_i32 = jnp.zeros((_L,), jnp.int32)
    ins = (x0_hbm, x1_hbm, x2_hbm)
    outs = (o0_hbm, o1_hbm, o2_hbm)

    @pl.when(wid < 3)
    def _():
        for w in range(3):
            @pl.when(wid == w)
            def _(w=w):
                pltpu.sync_copy(ins[w], xv)

        def zero_body(i, _):
            for u in range(8):
                hist[pl.ds((i * 8 + u) * _L, _L)] = z16_i32
            return 0

        lax.fori_loop(0, _HWORDS // (8 * _L), zero_body, 0)

        def pass_a(i, accs):
            vs = [xv[pl.ds((i * 4 + u) * _L, _L)] for u in range(4)]
            new, uks, poss = [], [], []
            for u in range(4):
                v = vs[u]
                e = jnp.exp(v - _C)
                s = 1.0 / (1.0 + e)
                s = jnp.clip(s, _EPS, 1.0 - _EPS)
                new.append(accs[u] + (1.0 - s))
                vc = jnp.where(v == 0.0, 0.0, v)
                bi = plsc.bitcast(vc, jnp.int32)
                flip = (bi >> 31) | jnp.int32(-0x80000000)
                uk = plsc.bitcast(bi ^ flip, jnp.uint32)
                uks.append(uk)
                poss.append(lane_base + (uk >> 24).astype(jnp.int32))
            for u in range(4):
                keys[pl.ds((i * 4 + u) * _L, _L)] = uks[u]
            for u in range(4):
                plsc.addupdate_scatter(hist, [poss[u]], ones_i32)
            return tuple(new)

        z16f = jnp.zeros((_L,), jnp.float32)
        a0, a1, a2, a3 = lax.fori_loop(0, _NV // 4, pass_a,
                                       (z16f, z16f, z16f, z16f))
        total = jnp.sum((a0 + a1) + (a2 + a3))
        k = _round_half_even(np.float32(_N) - total)

        bucket, below, _cnt = _pick_bucket(hist, k)
        count_lt = below
        prefix = bucket.astype(jnp.uint32)
        for lvl in range(1, 4):
            shift = jnp.uint32(24 - 8 * lvl)

            def scan_body(i, _, shift=shift, prefix=prefix):
                vs = [keys[pl.ds((i * 8 + u) * _L, _L)] for u in range(8)]
                poss, ms = [], []
                for u in range(8):
                    v = vs[u]
                    b = ((v >> shift) & jnp.uint32(0xFF)).astype(jnp.int32)
                    poss.append(lane_base + b)
                    ms.append((v >> (shift + jnp.uint32(8))) == prefix)
                for u in range(8):
                    plsc.addupdate_scatter(hist, [poss[u]], ones_i32,
                                           mask=ms[u])
                return 0

            lax.fori_loop(0, _NV // 8, scan_body, 0)
            bucket, below, cnt_eq = _pick_bucket(hist, k - count_lt)
            count_lt = count_lt + below
            prefix = (prefix << jnp.uint32(8)) | bucket.astype(jnp.uint32)
        t_key = prefix
        tie_budget = k - count_lt
        need_tie = cnt_eq > tie_budget
        tie_trips = jnp.where(need_tie, _NV // 8, 0)

        def tie_a(i, _):
            vs = [keys[pl.ds((i * 8 + u) * _L, _L)] for u in range(8)]
            poss, ms = [], []
            for u in range(8):
                idxv = lane + (i * 8 + u) * _L
                poss.append(lane_base + (idxv >> 5))
                ms.append(vs[u] == t_key)
            for u in range(8):
                plsc.addupdate_scatter(hist, [poss[u]], ones_i32, mask=ms[u])
            return 0

        lax.fori_loop(0, tie_trips, tie_a, 0)
        buck_a, below_a, _ca = _pick_bucket(hist, tie_budget)

        def tie_b(i, _):
            vs = [keys[pl.ds((i * 8 + u) * _L, _L)] for u in range(8)]
            poss, ms = [], []
            for u in range(8):
                idxv = lane + (i * 8 + u) * _L
                poss.append(lane_base + (idxv & 31))
                ms.append((vs[u] == t_key) & ((idxv >> 5) == buck_a))
            for u in range(8):
                plsc.addupdate_scatter(hist, [poss[u]], ones_i32, mask=ms[u])
            return 0

        lax.fori_loop(0, tie_trips, tie_b, 0)
        buck_b, _ub, _cb = _pick_bucket(hist, tie_budget - below_a)
        i_cut = jnp.where(need_tie, buck_a * 32 + buck_b, jnp.int32(_N))

        it_key = plsc.bitcast(
            jnp.full((_L,), t_key ^ jnp.uint32(0x80000000)), jnp.int32)

        def mask_body(i, _):
            vs = [keys[pl.ds((i * 4 + u) * _L, _L)] for u in range(4)]
            outs = []
            for u in range(4):
                v = vs[u]
                ik = plsc.bitcast(v ^ jnp.uint32(0x80000000), jnp.int32)
                eq = v == t_key
                idxv = lane + (i * 4 + u) * _L
                zero = (ik < it_key) | (eq & (idxv <= i_cut))
                outs.append(jnp.where(zero, 0.0, 1.0))
            for u in range(4):
                outv[pl.ds((i * 4 + u) * _L, _L)] = outs[u]
            return 0

        lax.fori_loop(0, _NV // 4, mask_body, 0)
        for w in range(3):
            @pl.when(wid == w)
            def _(w=w):
                pltpu.sync_copy(outv, outs[w])


@jax.jit
def _run_sc(x0, x1, x2):
    f = pl.kernel(
        _sc_body,
        out_type=(jax.ShapeDtypeStruct((_N,), jnp.float32),) * 3,
        mesh=plsc.VectorSubcoreMesh(core_axis_name="c", subcore_axis_name="s"),
        scratch_types=[
            pltpu.VMEM((_N,), jnp.float32),
            pltpu.VMEM((_N,), jnp.uint32),
            pltpu.VMEM((_HWORDS,), jnp.int32),
            pltpu.VMEM((_N,), jnp.float32),
        ],
        compiler_params=pltpu.CompilerParams(needs_layout_passes=False),
    )
    return f(x0, x1, x2)


def kernel(self_att_layer_loga, cross_att_layer_loga, ffn_layer_loga):
    return _run_sc(self_att_layer_loga, cross_att_layer_loga, ffn_layer_loga)


# speedup vs baseline: 3.0822x; 1.0016x over previous
import numpy as np
import jax
import jax.numpy as jnp
from jax import lax
from jax.experimental import pallas as pl
from jax.experimental.pallas import tpu as pltpu
from jax.experimental.pallas import tpu_sc as plsc

_EPS = 1e-6
_LIMIT_A = -0.1
_LIMIT_B = 1.1
_BETA = 2.0 / 3.0
_XN = (0.0 - _LIMIT_A) / (_LIMIT_B - _LIMIT_A)
_C = float(np.log(_XN / (1.0 - _XN))) * _BETA

_N = 8192
_L = 16
_NV = _N // _L
_NB = 256
_BIG = np.int32(0x7FFFFFFF)
_STRIDE = _NB + 1
_HWORDS = -(-(_STRIDE * _L) // 128) * 128


def _round_half_even(x):
    tr = x.astype(jnp.int32)
    frac = x - tr.astype(jnp.float32)
    bump = jnp.where(frac > 0.5, jnp.int32(1),
                     jnp.where(frac == 0.5, tr & 1, jnp.int32(0)))
    return tr + bump


def _pick_bucket(hist, kk):
    lane = lax.iota(jnp.int32, _L)
    z16 = jnp.zeros((_L,), jnp.int32)

    def body(g, carry):
        run_min, run_min2, cnt_carry = carry
        acc = z16
        for l in range(_L):
            sl = pl.ds(l * _STRIDE + g * _L, _L)
            acc = acc + hist[sl]
            hist[sl] = z16
        cum = plsc.cumsum(acc) + cnt_carry
        found = cum >= kk
        base = (g * _L + lane) << 14
        cand = jnp.where(found, base | (cum - acc), _BIG)
        cand2 = jnp.where(found, base | cum, _BIG)
        return (jnp.minimum(run_min, cand), jnp.minimum(run_min2, cand2),
                cnt_carry + jnp.sum(acc))

    run_min, run_min2, _ = lax.fori_loop(
        0, _NB // _L, body,
        (jnp.full((_L,), _BIG), jnp.full((_L,), _BIG), jnp.int32(0)))
    cm = jnp.min(run_min)
    below = cm & jnp.int32(0x3FFF)
    in_bucket = (jnp.min(run_min2) & jnp.int32(0x3FFF)) - below
    return cm >> 14, below, in_bucket


def _sc_body(x0_hbm, x1_hbm, x2_hbm, o0_hbm, o1_hbm, o2_hbm,
             xv, keys, hist, outv):
    wid = lax.axis_index("s") * 2 + lax.axis_index("c")
    lane = lax.iota(jnp.int32, _L)
    lane_base = lane * _STRIDE
    ones_i32 = jnp.ones((_L,), jnp.int32)
    z16_i32 = jnp.zeros((_L,), jnp.int32)
    ins = (x0_hbm, x1_hbm, x2_hbm)
    outs = (o0_hbm, o1_hbm, o2_hbm)

    @pl.when(wid < 3)
    def _():
        for w in range(3):
            @pl.when(wid == w)
            def _(w=w):
                pltpu.sync_copy(ins[w], xv)

        def zero_body(i, _):
            for u in range(8):
                hist[pl.ds((i * 8 + u) * _L, _L)] = z16_i32
            return 0

        lax.fori_loop(0, _HWORDS // (8 * _L), zero_body, 0)

        def pass_a(i, accs):
            vs = [xv[pl.ds((i * 4 + u) * _L, _L)] for u in range(4)]
            new, uks, poss = [], [], []
            for u in range(4):
                v = vs[u]
                e = jnp.exp(v - _C)
                s = 1.0 / (1.0 + e)
                s = jnp.clip(s, _EPS, 1.0 - _EPS)
                new.append(accs[u] + (1.0 - s))
                vc = jnp.where(v == 0.0, 0.0, v)
                bi = plsc.bitcast(vc, jnp.int32)
                flip = (bi >> 31) | jnp.int32(-0x80000000)
                uk = plsc.bitcast(bi ^ flip, jnp.uint32)
                uks.append(uk)
                poss.append(lane_base + (uk >> 24).astype(jnp.int32))
            for u in range(4):
                keys[pl.ds((i * 4 + u) * _L, _L)] = uks[u]
            for u in range(4):
                plsc.addupdate_scatter(hist, [poss[u]], ones_i32)
            return tuple(new)

        z16f = jnp.zeros((_L,), jnp.float32)
        a0, a1, a2, a3 = lax.fori_loop(0, _NV // 4, pass_a,
                                       (z16f, z16f, z16f, z16f))
        total = jnp.sum((a0 + a1) + (a2 + a3))
        k = _round_half_even(np.float32(_N) - total)

        bucket, below, _cnt = _pick_bucket(hist, k)
        count_lt = below
        prefix = bucket.astype(jnp.uint32)
        for lvl in range(1, 4):
            shift = jnp.uint32(24 - 8 * lvl)

            def scan_body(i, _, shift=shift, prefix=prefix):
                vs = [keys[pl.ds((i * 8 + u) * _L, _L)] for u in range(8)]
                poss, ms = [], []
                for u in range(8):
                    v = vs[u]
                    b = ((v >> shift) & jnp.uint32(0xFF)).astype(jnp.int32)
                    poss.append(lane_base + b)
                    ms.append((v >> (shift + jnp.uint32(8))) == prefix)
                for u in range(8):
                    plsc.addupdate_scatter(hist, [poss[u]], ones_i32,
                                           mask=ms[u])
                return 0

            lax.fori_loop(0, _NV // 8, scan_body, 0)
            bucket, below, cnt_eq = _pick_bucket(hist, k - count_lt)
            count_lt = count_lt + below
            prefix = (prefix << jnp.uint32(8)) | bucket.astype(jnp.uint32)
        t_key = prefix
        tie_budget = k - count_lt
        need_tie = cnt_eq > tie_budget
        tie_trips = jnp.where(need_tie, _NV // 8, 0)

        def tie_a(i, _):
            vs = [keys[pl.ds((i * 8 + u) * _L, _L)] for u in range(8)]
            poss, ms = [], []
            for u in range(8):
                idxv = lane + (i * 8 + u) * _L
                poss.append(lane_base + (idxv >> 5))
                ms.append(vs[u] == t_key)
            for u in range(8):
                plsc.addupdate_scatter(hist, [poss[u]], ones_i32, mask=ms[u])
            return 0

        lax.fori_loop(0, tie_trips, tie_a, 0)
        buck_a, below_a, _ca = _pick_bucket(hist, tie_budget)

        def tie_b(i, _):
            vs = [keys[pl.ds((i * 8 + u) * _L, _L)] for u in range(8)]
            poss, ms = [], []
            for u in range(8):
                idxv = lane + (i * 8 + u) * _L
                poss.append(lane_base + (idxv & 31))
                ms.append((vs[u] == t_key) & ((idxv >> 5) == buck_a))
            for u in range(8):
                plsc.addupdate_scatter(hist, [poss[u]], ones_i32, mask=ms[u])
            return 0

        lax.fori_loop(0, tie_trips, tie_b, 0)
        buck_b, _ub, _cb = _pick_bucket(hist, tie_budget - below_a)
        i_cut = jnp.where(need_tie, buck_a * 32 + buck_b, jnp.int32(_N))

        it_key = plsc.bitcast(
            jnp.full((_L,), t_key ^ jnp.uint32(0x80000000)), jnp.int32)

        def mask_body(i, _):
            vs = [keys[pl.ds((i * 4 + u) * _L, _L)] for u in range(4)]
            outs = []
            for u in range(4):
                v = vs[u]
                ik = plsc.bitcast(v ^ jnp.uint32(0x80000000), jnp.int32)
                eq = v == t_key
                idxv = lane + (i * 4 + u) * _L
                zero = (ik < it_key) | (eq & (idxv <= i_cut))
                outs.append(jnp.where(zero, 0.0, 1.0))
            for u in range(4):
                outv[pl.ds((i * 4 + u) * _L, _L)] = outs[u]
            return 0

        lax.fori_loop(0, _NV // 4, mask_body, 0)
        for w in range(3):
            @pl.when(wid == w)
            def _(w=w):
                pltpu.sync_copy(outv, outs[w])


@jax.jit
def _run_sc(x0, x1, x2):
    f = pl.kernel(
        _sc_body,
        out_type=(jax.ShapeDtypeStruct((_N,), jnp.float32),) * 3,
        mesh=plsc.VectorSubcoreMesh(core_axis_name="c", subcore_axis_name="s"),
        scratch_types=[
            pltpu.VMEM((_N,), jnp.float32),
            pltpu.VMEM((_N,), jnp.uint32),
            pltpu.VMEM((_HWORDS,), jnp.int32),
            pltpu.VMEM((_N,), jnp.float32),
        ],
        compiler_params=pltpu.CompilerParams(needs_layout_passes=False),
    )
    return f(x0, x1, x2)


def kernel(self_att_layer_loga, cross_att_layer_loga, ffn_layer_loga):
    return _run_sc(self_att_layer_loga, cross_att_layer_loga, ffn_layer_loga)


# speedup vs baseline: 3.1102x; 1.0091x over previous
import numpy as np
import jax
import jax.numpy as jnp
from jax import lax
from jax.experimental import pallas as pl
from jax.experimental.pallas import tpu as pltpu
from jax.experimental.pallas import tpu_sc as plsc

_EPS = 1e-6
_LIMIT_A = -0.1
_LIMIT_B = 1.1
_BETA = 2.0 / 3.0
_XN = (0.0 - _LIMIT_A) / (_LIMIT_B - _LIMIT_A)
_C = float(np.log(_XN / (1.0 - _XN))) * _BETA

_N = 8192
_L = 16
_NV = _N // _L
_NB = 256
_BIG = np.int32(0x7FFFFFFF)
_STRIDE = _NB + 1
_HWORDS = -(-(_STRIDE * _L) // 128) * 128


def _round_half_even(x):
    tr = x.astype(jnp.int32)
    frac = x - tr.astype(jnp.float32)
    bump = jnp.where(frac > 0.5, jnp.int32(1),
                     jnp.where(frac == 0.5, tr & 1, jnp.int32(0)))
    return tr + bump


def _pick_bucket(hist, kk):
    lane = lax.iota(jnp.int32, _L)
    z16 = jnp.zeros((_L,), jnp.int32)

    def body(g, carry):
        run_min, run_min2, cnt_carry = carry
        acc = z16
        for l in range(_L):
            sl = pl.ds(l * _STRIDE + g * _L, _L)
            acc = acc + hist[sl]
            hist[sl] = z16
        cum = plsc.cumsum(acc) + cnt_carry
        found = cum >= kk
        base = (g * _L + lane) << 14
        cand = jnp.where(found, base | (cum - acc), _BIG)
        cand2 = jnp.where(found, base | cum, _BIG)
        return (jnp.minimum(run_min, cand), jnp.minimum(run_min2, cand2),
                cnt_carry + jnp.sum(acc))

    run_min, run_min2, _ = lax.fori_loop(
        0, _NB // _L, body,
        (jnp.full((_L,), _BIG), jnp.full((_L,), _BIG), jnp.int32(0)))
    cm = jnp.min(run_min)
    below = cm & jnp.int32(0x3FFF)
    in_bucket = (jnp.min(run_min2) & jnp.int32(0x3FFF)) - below
    return cm >> 14, below, in_bucket


def _sc_body(x0_hbm, x1_hbm, x2_hbm, o0_hbm, o1_hbm, o2_hbm,
             xv, keys, hist, outv):
    wid = lax.axis_index("s") * 2 + lax.axis_index("c")
    lane = lax.iota(jnp.int32, _L)
    lane_base = lane * _STRIDE
    ones_i32 = jnp.ones((_L,), jnp.int32)
    z16_i32 = jnp.zeros((_L,), jnp.int32)
    ins = (x0_hbm, x1_hbm, x2_hbm)
    outs = (o0_hbm, o1_hbm, o2_hbm)

    @pl.when(wid < 3)
    def _():
        for w in range(3):
            @pl.when(wid == w)
            def _(w=w):
                pltpu.sync_copy(ins[w], xv)

        def zero_body(i, _):
            for u in range(8):
                hist[pl.ds((i * 8 + u) * _L, _L)] = z16_i32
            return 0

        lax.fori_loop(0, _HWORDS // (8 * _L), zero_body, 0)

        def pass_a(i, accs):
            vs = [xv[pl.ds((i * 8 + u) * _L, _L)] for u in range(8)]
            new, uks, poss = [], [], []
            for u in range(8):
                v = vs[u]
                e = jnp.exp(v - _C)
                s = 1.0 / (1.0 + e)
                s = jnp.clip(s, _EPS, 1.0 - _EPS)
                new.append(accs[u] + (1.0 - s))
                vc = jnp.where(v == 0.0, 0.0, v)
                bi = plsc.bitcast(vc, jnp.int32)
                flip = (bi >> 31) | jnp.int32(-0x80000000)
                uk = plsc.bitcast(bi ^ flip, jnp.uint32)
                uks.append(uk)
                poss.append(lane_base + (uk >> 24).astype(jnp.int32))
            for u in range(8):
                keys[pl.ds((i * 8 + u) * _L, _L)] = uks[u]
            for u in range(8):
                plsc.addupdate_scatter(hist, [poss[u]], ones_i32)
            return tuple(new)

        z16f = jnp.zeros((_L,), jnp.float32)
        accs = lax.fori_loop(0, _NV // 8, pass_a, (z16f,) * 8)
        total = jnp.sum(((accs[0] + accs[1]) + (accs[2] + accs[3]))
                        + ((accs[4] + accs[5]) + (accs[6] + accs[7])))
        k = _round_half_even(np.float32(_N) - total)

        bucket, below, _cnt = _pick_bucket(hist, k)
        count_lt = below
        prefix = bucket.astype(jnp.uint32)
        for lvl in range(1, 4):
            shift = jnp.uint32(24 - 8 * lvl)

            def scan_body(i, _, shift=shift, prefix=prefix):
                vs = [keys[pl.ds((i * 8 + u) * _L, _L)] for u in range(8)]
                poss, ms = [], []
                for u in range(8):
                    v = vs[u]
                    b = ((v >> shift) & jnp.uint32(0xFF)).astype(jnp.int32)
                    poss.append(lane_base + b)
                    ms.append((v >> (shift + jnp.uint32(8))) == prefix)
                for u in range(8):
                    plsc.addupdate_scatter(hist, [poss[u]], ones_i32,
                                           mask=ms[u])
                return 0

            lax.fori_loop(0, _NV // 8, scan_body, 0)
            bucket, below, cnt_eq = _pick_bucket(hist, k - count_lt)
            count_lt = count_lt + below
            prefix = (prefix << jnp.uint32(8)) | bucket.astype(jnp.uint32)
        t_key = prefix
        tie_budget = k - count_lt
        need_tie = cnt_eq > tie_budget
        tie_trips = jnp.where(need_tie, _NV // 8, 0)

        def tie_a(i, _):
            vs = [keys[pl.ds((i * 8 + u) * _L, _L)] for u in range(8)]
            poss, ms = [], []
            for u in range(8):
                idxv = lane + (i * 8 + u) * _L
                poss.append(lane_base + (idxv >> 5))
                ms.append(vs[u] == t_key)
            for u in range(8):
                plsc.addupdate_scatter(hist, [poss[u]], ones_i32, mask=ms[u])
            return 0

        lax.fori_loop(0, tie_trips, tie_a, 0)
        buck_a, below_a, _ca = _pick_bucket(hist, tie_budget)

        def tie_b(i, _):
            vs = [keys[pl.ds((i * 8 + u) * _L, _L)] for u in range(8)]
            poss, ms = [], []
            for u in range(8):
                idxv = lane + (i * 8 + u) * _L
                poss.append(lane_base + (idxv & 31))
                ms.append((vs[u] == t_key) & ((idxv >> 5) == buck_a))
            for u in range(8):
                plsc.addupdate_scatter(hist, [poss[u]], ones_i32, mask=ms[u])
            return 0

        lax.fori_loop(0, tie_trips, tie_b, 0)
        buck_b, _ub, _cb = _pick_bucket(hist, tie_budget - below_a)
        i_cut = jnp.where(need_tie, buck_a * 32 + buck_b, jnp.int32(_N))

        it_key = plsc.bitcast(
            jnp.full((_L,), t_key ^ jnp.uint32(0x80000000)), jnp.int32)

        def mask_body(i, _):
            vs = [keys[pl.ds((i * 4 + u) * _L, _L)] for u in range(4)]
            outs = []
            for u in range(4):
                v = vs[u]
                ik = plsc.bitcast(v ^ jnp.uint32(0x80000000), jnp.int32)
                eq = v == t_key
                idxv = lane + (i * 4 + u) * _L
                zero = (ik < it_key) | (eq & (idxv <= i_cut))
                outs.append(jnp.where(zero, 0.0, 1.0))
            for u in range(4):
                outv[pl.ds((i * 4 + u) * _L, _L)] = outs[u]
            return 0

        lax.fori_loop(0, _NV // 4, mask_body, 0)
        for w in range(3):
            @pl.when(wid == w)
            def _(w=w):
                pltpu.sync_copy(outv, outs[w])


@jax.jit
def _run_sc(x0, x1, x2):
    f = pl.kernel(
        _sc_body,
        out_type=(jax.ShapeDtypeStruct((_N,), jnp.float32),) * 3,
        mesh=plsc.VectorSubcoreMesh(core_axis_name="c", subcore_axis_name="s"),
        scratch_types=[
            pltpu.VMEM((_N,), jnp.float32),
            pltpu.VMEM((_N,), jnp.uint32),
            pltpu.VMEM((_HWORDS,), jnp.int32),
            pltpu.VMEM((_N,), jnp.float32),
        ],
        compiler_params=pltpu.CompilerParams(needs_layout_passes=False),
    )
    return f(x0, x1, x2)


def kernel(self_att_layer_loga, cross_att_layer_loga, ffn_layer_loga):
    return _run_sc(self_att_layer_loga, cross_att_layer_loga, ffn_layer_loga)
